# baseline (speedup 1.0000x reference)
import functools

import jax
import jax.numpy as jnp
from jax import lax
from jax.experimental import pallas as pl
from jax.experimental.pallas import tpu as pltpu
from jax.experimental.pallas import tpu_sc as plsc

M = 65664
K1 = 1025

NC = 2
NS = 16
NW = NC * NS
G = 32
NGRP = M // G
NG_MAIN = NGRP // NW
NG_EXTRA = NGRP - NG_MAIN * NW
IDX_W = (NG_MAIN + 1) * G
NBUF = 2


def _tc_argmax(alpha, gumbels):
    BM = 1024

    def body(a_ref, g_ref, o_ref):
        x = a_ref[...] + g_ref[...]
        m = jnp.max(x, axis=-1, keepdims=True)
        col = lax.broadcasted_iota(jnp.int32, x.shape, 1)
        o_ref[...] = jnp.min(jnp.where(x == m, col, K1), axis=-1, keepdims=True)

    return pl.pallas_call(
        body,
        grid=(pl.cdiv(M, BM),),
        in_specs=[
            pl.BlockSpec((BM, K1), lambda i: (i, 0)),
            pl.BlockSpec((BM, K1), lambda i: (i, 0)),
        ],
        out_specs=pl.BlockSpec((BM, 1), lambda i: (i, 0)),
        out_shape=jax.ShapeDtypeStruct((M, 1), jnp.int32),
    )(alpha, gumbels)


def _scatter_group(buf, idx_v, start, value):
    lanes = lax.iota(jnp.int32, 16)
    for sub in range(G // 16):
        iv = idx_v[pl.ds(start + sub * 16, 16)]
        plsc.store_scatter(buf, [lanes + sub * 16, iv], value)


def _sc_onehot(idx_flat):
    mesh = plsc.VectorSubcoreMesh(core_axis_name="c", subcore_axis_name="s")

    @functools.partial(
        pl.kernel,
        out_type=jax.ShapeDtypeStruct((M, K1), jnp.float32),
        mesh=mesh,
        scratch_types=(
            [pltpu.VMEM((IDX_W,), jnp.int32)]
            + [pltpu.VMEM((G, K1), jnp.float32) for _ in range(NBUF)]
            + [pltpu.SemaphoreType.DMA for _ in range(NBUF)]
        ),
        compiler_params=pltpu.CompilerParams(
            use_tc_tiling_on_sc=False, needs_layout_passes=False
        ),
    )
    def run(idx_hbm, out_hbm, idx_v, *bufs_and_sems):
        bufs = bufs_and_sems[:NBUF]
        sems = bufs_and_sems[NBUF:]
        wid = lax.axis_index("c") * NS + lax.axis_index("s")
        ones = jnp.ones((16,), jnp.float32)
        zeros = jnp.zeros((16,), jnp.float32)

        pltpu.sync_copy(idx_hbm.at[pl.ds(wid * IDX_W, IDX_W)], idx_v)

        for buf in bufs:
            def zero_row(r, _, buf=buf):
                for c in range(K1 // 16):
                    buf[r, pl.ds(c * 16, 16)] = zeros
                buf[r, pl.ds(K1 - 16, 16)] = zeros
                return 0

            lax.fori_loop(0, G, zero_row, 0)

        def row0_of(j):
            return pl.multiple_of(G * NW * j + G * wid, G)

        def fire(b, j):
            _scatter_group(bufs[b], idx_v, j * G, ones)
            pltpu.async_copy(bufs[b], out_hbm.at[pl.ds(row0_of(j), G)], sems[b])

        def drain(b, j_prev):
            pltpu.make_async_copy(
                bufs[b], out_hbm.at[pl.ds(row0_of(j_prev), G)], sems[b]
            ).wait()
            _scatter_group(bufs[b], idx_v, j_prev * G, zeros)

        for b in range(NBUF):
            fire(b, b)

        def outer(i, _):
            for b in range(NBUF):
                j = NBUF + i * NBUF + b
                drain(b, j - NBUF)
                fire(b, j)
            return 0

        lax.fori_loop(0, (NG_MAIN - NBUF) // NBUF, outer, 0)

        for b in range(NBUF):
            drain(b, NG_MAIN - NBUF + b)

        @pl.when(wid < NG_EXTRA)
        def _():
            fire(0, NG_MAIN)
            pltpu.make_async_copy(
                bufs[0], out_hbm.at[pl.ds(row0_of(NG_MAIN), G)], sems[0]
            ).wait()

    return run(idx_flat)


def kernel(alpha, gumbels, tau):
    del tau
    idx = _tc_argmax(alpha, gumbels)[:, 0]
    idx_pad = jnp.pad(idx, (0, NW * IDX_W - M))
    idx_flat = (
        idx_pad.reshape(NG_MAIN + 1, NW, G).transpose(1, 0, 2).reshape(-1)
    )
    return _sc_onehot(idx_flat)


# speedup vs baseline: 5.0198x; 5.0198x over previous
import functools

import jax
import jax.numpy as jnp
from jax import lax
from jax.experimental import pallas as pl
from jax.experimental.pallas import tpu as pltpu
from jax.experimental.pallas import tpu_sc as plsc

M = 65664
K1 = 1025

NC = 2
NS = 16
NW = NC * NS
LT = 128
NT = M // LT
TPW = NT // NW
BN = 1024
GRID = pl.cdiv(M, BN)
IDX_ROWS = GRID * BN // LT

RC = ((0, 256), (256, 256), (512, 256), (768, 257))


def _tc_argmax_t(at, gt):
    def body(a_ref, g_ref, o_ref):
        x = a_ref[...] + g_ref[...]
        m = jnp.max(x, axis=0, keepdims=True)
        row = lax.broadcasted_iota(jnp.int32, x.shape, 0)
        o_ref[...] = jnp.min(jnp.where(x == m, row, K1), axis=0)[None, None, :]

    return pl.pallas_call(
        body,
        grid=(GRID,),
        in_specs=[
            pl.BlockSpec((K1, BN), lambda i: (0, i)),
            pl.BlockSpec((K1, BN), lambda i: (0, i)),
        ],
        out_specs=pl.BlockSpec((1, 1, BN), lambda i: (i, 0, 0)),
        out_shape=jax.ShapeDtypeStruct((GRID, 1, BN), jnp.int32),
    )(at, gt)


def _sc_onehot_t(idx2d):
    mesh = plsc.VectorSubcoreMesh(core_axis_name="c", subcore_axis_name="s")

    @functools.partial(
        pl.kernel,
        out_type=jax.ShapeDtypeStruct((K1, M), jnp.float32),
        mesh=mesh,
        scratch_types=(
            pltpu.VMEM((TPW + 8, LT), jnp.int32),
            pltpu.VMEM((RC[0][1], LT), jnp.float32),
            pltpu.VMEM((RC[3][1], LT), jnp.float32),
            pltpu.SemaphoreType.DMA,
            pltpu.SemaphoreType.DMA,
        ),
        compiler_params=pltpu.CompilerParams(
            use_tc_tiling_on_sc=True, needs_layout_passes=False
        ),
    )
    def run(idx_hbm, out_hbm, idx_v, buf0, buf1, sem0, sem1):
        wid = lax.axis_index("c") * NS + lax.axis_index("s")
        lanes = lax.iota(jnp.int32, 16)
        ones = jnp.ones((16,), jnp.float32)
        zeros = jnp.zeros((16,), jnp.float32)
        bufs = (buf0, buf1)
        sems = (sem0, sem1)

        pltpu.sync_copy(idx_hbm.at[pl.ds(wid * TPW, TPW)], idx_v.at[pl.ds(0, TPW)])

        @pl.when(wid == 0)
        def _():
            pltpu.sync_copy(
                idx_hbm.at[pl.ds(NW * TPW, 8)], idx_v.at[pl.ds(TPW, 8)]
            )

        for buf in bufs:
            def zero_row(r, _, buf=buf):
                for c in range(LT // 16):
                    buf[r, pl.ds(c * 16, 16)] = zeros
                return 0

            lax.fori_loop(0, buf.shape[0], zero_row, 0)

        def scat(b, jl, rc, value):
            r0, nr = RC[rc]
            for sub in range(LT // 16):
                iv = idx_v[jl, pl.ds(sub * 16, 16)]
                m = (iv >= r0) & (iv < r0 + nr)
                plsc.store_scatter(
                    bufs[b], [iv - r0, lanes + sub * 16], value, mask=m
                )

        def dma_refs(b, t, rc):
            r0, nr = RC[rc]
            src = bufs[b] if nr == bufs[b].shape[0] else bufs[b].at[pl.ds(0, nr)]
            dst = out_hbm.at[pl.ds(r0, nr), pl.ds(pl.multiple_of(t * LT, LT), LT)]
            return src, dst

        def fire(b, jl, t, rc):
            scat(b, jl, rc, ones)
            src, dst = dma_refs(b, t, rc)
            pltpu.async_copy(src, dst, sems[b])

        def drain(b, jl_prev, t_prev, rc_prev):
            src, dst = dma_refs(b, t_prev, rc_prev)
            pltpu.make_async_copy(src, dst, sems[b]).wait()
            scat(b, jl_prev, rc_prev, zeros)

        t0 = wid * TPW

        fire(0, 0, t0, 0)
        fire(1, 0, t0, 1)
        drain(0, 0, t0, 0)
        fire(0, 0, t0, 2)
        drain(1, 0, t0, 1)
        fire(1, 0, t0, 3)

        def tile_body(jl, _):
            t = t0 + jl
            drain(0, jl - 1, t - 1, 2)
            fire(0, jl, t, 0)
            drain(1, jl - 1, t - 1, 3)
            fire(1, jl, t, 1)
            drain(0, jl, t, 0)
            fire(0, jl, t, 2)
            drain(1, jl, t, 1)
            fire(1, jl, t, 3)
            return 0

        lax.fori_loop(1, TPW, tile_body, 0)

        @pl.when(wid == 0)
        def _():
            t = NW * TPW
            drain(0, TPW - 1, t0 + TPW - 1, 2)
            fire(0, TPW, t, 0)
            drain(1, TPW - 1, t0 + TPW - 1, 3)
            fire(1, TPW, t, 1)
            drain(0, TPW, t, 0)
            fire(0, TPW, t, 2)
            drain(1, TPW, t, 1)
            fire(1, TPW, t, 3)

        t_last = jnp.where(wid == 0, NW * TPW, t0 + TPW - 1)
        for b, rc in ((0, 2), (1, 3)):
            src, dst = dma_refs(b, t_last, rc)
            pltpu.make_async_copy(src, dst, sems[b]).wait()

    return run(idx2d)


def kernel(alpha, gumbels, tau):
    del tau
    idx3 = _tc_argmax_t(alpha.T, gumbels.T)
    idx2d = idx3.reshape(IDX_ROWS, LT)
    return _sc_onehot_t(idx2d).T
